# Initial kernel scaffold; baseline (speedup 1.0000x reference)
import jax
import jax.numpy as jnp
from jax.experimental import pallas as pl


def kernel(h, edge_index, x, W_self1, W_neigh1, b1, W_self2, W_neigh2, b2, W_out, b_out):
    raise NotImplementedError("write your pallas kernel here")


# speedup vs baseline: 4.0473x; 4.0473x over previous
import functools

import jax
import jax.numpy as jnp
from jax import lax
from jax.experimental import pallas as pl
from jax.experimental.pallas import tpu as pltpu
from jax.experimental.pallas import tpu_sc as plsc

_NC = 2
_NS = 16
_NW = _NC * _NS
_CH = 128


def _sc_aggregate(table, srcp, dstp, zrows, zdeg, ones, n_pad, with_deg):
  E_pad = srcp.shape[0]
  D = table.shape[1]
  epw = E_pad // _NW
  n_chunks = epw // _CH
  rpt = n_pad // _NS
  mesh = plsc.VectorSubcoreMesh(core_axis_name="c", subcore_axis_name="s",
                                num_cores=_NC, num_subcores=_NS)
  out_type = [jax.ShapeDtypeStruct((_NC * n_pad, D), jnp.float32)]
  scratch = [
      pltpu.VMEM((_CH,), jnp.int32),
      pltpu.VMEM((_CH,), jnp.int32),
      pltpu.VMEM((_CH, D), jnp.float32),
      pltpu.VMEM_SHARED((n_pad, D), jnp.float32),
      pltpu.SemaphoreType.DMA,
  ]
  if with_deg:
    out_type.append(jax.ShapeDtypeStruct((_NC * n_pad,), jnp.float32))
    scratch += [pltpu.VMEM((_CH,), jnp.float32),
                pltpu.VMEM_SHARED((n_pad,), jnp.float32)]

  def body(*refs):
    if with_deg:
      (table_h, src_h, dst_h, zr_h, zd_h, on_h, acc_h, deg_h,
       src_v, dst_v, rows_v, acc_sh, sem, ones_v, deg_sh) = refs
    else:
      (table_h, src_h, dst_h, zr_h,
       acc_h, src_v, dst_v, rows_v, acc_sh, sem) = refs
    cid = lax.axis_index("c")
    sid = lax.axis_index("s")
    w = cid * _NS + sid

    r0 = sid * rpt
    pltpu.sync_copy(zr_h, acc_sh.at[pl.ds(r0, rpt)])
    if with_deg:
      pltpu.sync_copy(zd_h, deg_sh.at[pl.ds(r0, rpt)])
      pltpu.sync_copy(on_h, ones_v)
    plsc.subcore_barrier()

    base_w = w * epw

    def step(i, carry):
      base = base_w + i * _CH
      pltpu.sync_copy(src_h.at[pl.ds(base, _CH)], src_v)
      pltpu.sync_copy(dst_h.at[pl.ds(base, _CH)], dst_v)
      pltpu.async_copy(table_h.at[src_v], rows_v, sem).wait()
      pltpu.sync_copy(rows_v, acc_sh.at[dst_v], add=True)
      if with_deg:
        pltpu.sync_copy(ones_v, deg_sh.at[dst_v], add=True)
      return carry

    lax.fori_loop(0, n_chunks, step, 0)

    plsc.subcore_barrier()
    out_r0 = cid * n_pad + sid * rpt
    pltpu.sync_copy(acc_sh.at[pl.ds(r0, rpt)], acc_h.at[pl.ds(out_r0, rpt)])
    if with_deg:
      pltpu.sync_copy(deg_sh.at[pl.ds(r0, rpt)], deg_h.at[pl.ds(out_r0, rpt)])

  kfn = pl.kernel(body, out_type=tuple(out_type), mesh=mesh,
                  scratch_types=tuple(scratch))
  if with_deg:
    return kfn(table, srcp, dstp, zrows, zdeg, ones)
  return (kfn(table, srcp, dstp, zrows)[0],)


def _tc_dense(h_pad, acc, deg2, Ws, Wn, b, Wtail, btail):
  n_pad, D = h_pad.shape
  H = Ws.shape[1]
  BLK = 1024
  nb = n_pad // BLK

  def body(*refs):
    if Wtail is not None:
      (h_ref, a0_ref, a1_ref, d0_ref, d1_ref, ws_ref, wn_ref, b_ref,
       wt_ref, bt_ref, o_ref) = refs
    else:
      (h_ref, a0_ref, a1_ref, d0_ref, d1_ref, ws_ref, wn_ref, b_ref,
       o_ref) = refs
    inv = 1.0 / jnp.maximum(d0_ref[...] + d1_ref[...], 1.0)
    neigh = (a0_ref[...] + a1_ref[...]) * inv
    pre = (jnp.dot(h_ref[...], ws_ref[...], preferred_element_type=jnp.float32)
           + jnp.dot(neigh, wn_ref[...], preferred_element_type=jnp.float32)
           + b_ref[...])
    hh = jnp.maximum(pre, 0.0)
    if Wtail is not None:
      o_ref[...] = (jnp.dot(hh, wt_ref[...],
                            preferred_element_type=jnp.float32) + bt_ref[...])
    else:
      o_ref[...] = hh

  in_specs = [
      pl.BlockSpec((BLK, D), lambda i: (i, 0)),
      pl.BlockSpec((BLK, D), lambda i: (i, 0)),
      pl.BlockSpec((BLK, D), lambda i: (i + nb, 0)),
      pl.BlockSpec((BLK, 1), lambda i: (i, 0)),
      pl.BlockSpec((BLK, 1), lambda i: (i + nb, 0)),
      pl.BlockSpec((D, H), lambda i: (0, 0)),
      pl.BlockSpec((D, H), lambda i: (0, 0)),
      pl.BlockSpec((1, H), lambda i: (0, 0)),
  ]
  args = [h_pad, acc, acc, deg2, deg2, Ws, Wn, b.reshape(1, H)]
  out_w = H
  if Wtail is not None:
    in_specs += [pl.BlockSpec((H, Wtail.shape[1]), lambda i: (0, 0)),
                 pl.BlockSpec((1, Wtail.shape[1]), lambda i: (0, 0))]
    args += [Wtail, btail.reshape(1, -1)]
    out_w = Wtail.shape[1]
  return pl.pallas_call(
      body,
      grid=(nb,),
      in_specs=in_specs,
      out_specs=pl.BlockSpec((BLK, out_w), lambda i: (i, 0)),
      out_shape=jax.ShapeDtypeStruct((n_pad, out_w), jnp.float32),
  )(*args)


def _sc_pair(z_full, x0p, x1p, lidx):
  H = z_full.shape[1]
  P = x0p.shape[0]
  ppc = P // _NC
  ppw = P // _NW
  mesh = plsc.VectorSubcoreMesh(core_axis_name="c", subcore_axis_name="s",
                                num_cores=_NC, num_subcores=_NS)

  @functools.partial(
      pl.kernel, mesh=mesh,
      out_type=jax.ShapeDtypeStruct((P, H), jnp.float32),
      scratch_types=(
          pltpu.VMEM((ppw,), jnp.int32),
          pltpu.VMEM((ppw,), jnp.int32),
          pltpu.VMEM((ppw, H), jnp.float32),
          pltpu.VMEM_SHARED((ppc, H), jnp.float32),
          pltpu.SemaphoreType.DMA,
      ))
  def k(z_h, x0_h, x1_h, li_h, o_h, idx_v, dst_v, rows_v, acc_sh, sem):
    cid = lax.axis_index("c")
    sid = lax.axis_index("s")
    l0 = sid * ppw
    g0 = cid * ppc + l0
    pltpu.sync_copy(li_h.at[pl.ds(l0, ppw)], dst_v)
    pltpu.sync_copy(x0_h.at[pl.ds(g0, ppw)], idx_v)
    pltpu.async_copy(z_h.at[idx_v], rows_v, sem).wait()
    pltpu.sync_copy(rows_v, acc_sh.at[pl.ds(l0, ppw)])
    pltpu.sync_copy(x1_h.at[pl.ds(g0, ppw)], idx_v)
    pltpu.async_copy(z_h.at[idx_v], rows_v, sem).wait()
    pltpu.sync_copy(rows_v, acc_sh.at[dst_v], add=True)
    plsc.subcore_barrier()
    pltpu.sync_copy(acc_sh.at[pl.ds(l0, ppw)], o_h.at[pl.ds(g0, ppw)])

  return k(z_full, x0p, x1p, lidx)


def kernel(h, edge_index, x, W_self1, W_neigh1, b1, W_self2, W_neigh2, b2,
           W_out, b_out):
  N, D = h.shape
  E = edge_index.shape[1]
  H = W_self1.shape[1]
  C = W_out.shape[1]
  P = x.shape[0]

  n_pad = (N // 1024 + 1) * 1024
  grain = _NW * _CH
  e_pad = -(-E // grain) * grain

  src = edge_index[0]
  dst = edge_index[1]
  srcp = jnp.concatenate([src, jnp.zeros((e_pad - E,), jnp.int32)])
  dstp = jnp.concatenate([dst, jnp.full((e_pad - E,), n_pad - 1, jnp.int32)])

  rpt = n_pad // _NS
  zrows = jnp.zeros((rpt, D), jnp.float32)
  zdeg = jnp.zeros((rpt,), jnp.float32)
  ones = jnp.ones((_CH,), jnp.float32)

  h_pad = jnp.concatenate([h, jnp.zeros((n_pad - N, D), jnp.float32)])

  acc1, deg = _sc_aggregate(h_pad, srcp, dstp, zrows, zdeg, ones, n_pad,
                            with_deg=True)
  deg2 = deg.reshape(_NC * n_pad, 1)
  h1 = _tc_dense(h_pad, acc1, deg2, W_self1, W_neigh1, b1, None, None)

  (acc2,) = _sc_aggregate(h1, srcp, dstp, zrows, zdeg, ones, n_pad,
                          with_deg=False)
  W_out_pad = jnp.pad(W_out, ((0, 0), (0, H - C)))
  b_out_pad = jnp.pad(0.5 * b_out, (0, H - C))
  z_full = _tc_dense(h1, acc2, deg2, W_self2, W_neigh2, b2,
                     W_out_pad, b_out_pad)
  lidx = jnp.arange(P // _NC, dtype=jnp.int32)
  pair_full = _sc_pair(z_full, x[:, 0], x[:, 1], lidx)
  return pair_full[:, :C]
